# baseline (speedup 1.0000x reference)
import functools

import jax
import jax.numpy as jnp
from jax import lax
from jax.experimental import pallas as pl
from jax.experimental.pallas import tpu as pltpu
from jax.experimental.pallas import tpu_sc as plsc

VOCAB = 1000000
DIM = 64
HID = 128
OUT = 5
S = 200
B = 4096

NC = 2
NS = 16
NW = NC * NS
BPW = B // NW
LANES = 16
NBUF = 2

_mesh = plsc.VectorSubcoreMesh(core_axis_name="c", subcore_axis_name="s")


@functools.partial(
    pl.kernel,
    out_type=jax.ShapeDtypeStruct((B, DIM), jnp.float32),
    mesh=_mesh,
    scratch_types=[
        pltpu.VMEM((S, BPW), jnp.int32),
        pltpu.VMEM((NBUF, BPW, DIM), jnp.float32),
        pltpu.VMEM((BPW, DIM), jnp.float32),
        pltpu.SemaphoreType.DMA,
        pltpu.SemaphoreType.DMA,
    ],
    compiler_params=pltpu.CompilerParams(use_tc_tiling_on_sc=False),
)
def _pool_sum(text_hbm, table_hbm, out_hbm, idx_v, rows_v, acc_v, sem0, sem1):
    sems = (sem0, sem1)
    wid = lax.axis_index("s") * NC + lax.axis_index("c")
    base = wid * BPW

    pltpu.sync_copy(text_hbm.at[:, pl.ds(base, BPW)], idx_v)

    @plsc.parallel_loop(0, BPW, unroll=4)
    def _zero(r):
        for c in range(DIM // LANES):
            acc_v[r, pl.ds(c * LANES, LANES)] = jnp.zeros((LANES,), jnp.float32)

    def _issue(s, b):
        pltpu.async_copy(table_hbm.at[idx_v.at[s]], rows_v.at[b], sems[b])

    def _wait(b):
        pltpu.make_async_copy(
            table_hbm.at[idx_v.at[0]], rows_v.at[b], sems[b]
        ).wait()

    def _accum(b):
        @plsc.parallel_loop(0, BPW, unroll=2)
        def _body(r):
            for c in range(DIM // LANES):
                v = rows_v[b, r, pl.ds(c * LANES, LANES)]
                plsc.addupdate(acc_v.at[r, pl.ds(c * LANES, LANES)], v)

    for b in range(NBUF):
        _issue(b, b)

    def body(i, carry):
        for b in range(NBUF):
            s = NBUF * i + b
            _wait(b)
            _issue(s + NBUF, b)
            _accum(b)
        return carry

    lax.fori_loop(0, S // NBUF - 1, body, 0, unroll=False)

    for b in range(NBUF):
        _wait(b)
        _accum(b)

    pltpu.sync_copy(acc_v, out_hbm.at[pl.ds(base, BPW)])


def _mlp_body(x_ref, w1_ref, b1_ref, w2_ref, b2_ref, o_ref):
    x = x_ref[...] * (1.0 / S)
    h = lax.dot_general(
        x, w1_ref[...], (((1,), (1,)), ((), ())),
        preferred_element_type=jnp.float32,
        precision=lax.Precision.HIGHEST,
    )
    h = h + b1_ref[...]
    z = lax.dot_general(
        h, w2_ref[...], (((1,), (1,)), ((), ())),
        preferred_element_type=jnp.float32,
        precision=lax.Precision.HIGHEST,
    )
    z = z + b2_ref[...]
    z = z - jnp.max(z, axis=1, keepdims=True)
    e = jnp.exp(z)
    o_ref[...] = e / jnp.sum(e, axis=1, keepdims=True)


def _mlp(pooled_sum, W1, b1, W2, b2):
    return pl.pallas_call(
        _mlp_body,
        out_shape=jax.ShapeDtypeStruct((B, OUT), jnp.float32),
    )(pooled_sum, W1, b1.reshape(1, HID), W2, b2.reshape(1, OUT))


def kernel(text, table, W1, b1, W2, b2):
    pooled_sum = _pool_sum(text, table)
    return _mlp(pooled_sum, W1, b1, W2, b2)
